# baseline (speedup 1.0000x reference)
import functools

import jax
import jax.numpy as jnp
from jax import lax
from jax.experimental import pallas as pl
from jax.experimental.pallas import tpu as pltpu
from jax.experimental.pallas import tpu_sc as plsc

NUM_CORES = 2
NUM_SUBCORES = 16
NUM_WORKERS = NUM_CORES * NUM_SUBCORES
LANES = 16

EMBED_DIM = 64
CHUNK = 128
N_CHUNKS = 4
ROWS_PER_WORKER = CHUNK * N_CHUNKS


def _sc_body(in_tab, out_tab, idx_in, idx_pos, idx_neg,
             pos_out, neg_out,
             idxv_in, idxv_pos, idxv_neg,
             rows_in, rows_pos, rows_neg,
             pacc_v, nacc_v,
             score_pos, score_neg, sem):
    wid = lax.axis_index("s") * NUM_CORES + lax.axis_index("c")
    rbase = wid * N_CHUNKS

    pltpu.sync_copy(idx_in.at[pl.ds(rbase, N_CHUNKS)], idxv_in)
    pltpu.sync_copy(idx_pos.at[pl.ds(rbase, N_CHUNKS)], idxv_pos)
    pltpu.sync_copy(idx_neg.at[pl.ds(rbase, N_CHUNKS)], idxv_neg)

    copies = []
    for j in range(N_CHUNKS):
        sl = pl.ds(j * CHUNK, CHUNK)
        copies.append(pltpu.async_copy(in_tab.at[idxv_in.at[j]], rows_in.at[sl], sem))
        copies.append(pltpu.async_copy(out_tab.at[idxv_pos.at[j]], rows_pos.at[sl], sem))
        copies.append(pltpu.async_copy(out_tab.at[idxv_neg.at[j]], rows_neg.at[sl], sem))
    for c in copies:
        c.wait()

    iota16 = lax.iota(jnp.int32, LANES)

    def chunk_body(c, carry):
        for i in range(LANES):
            r = c * LANES + i
            accp = jnp.zeros((LANES,), jnp.float32)
            accn = jnp.zeros((LANES,), jnp.float32)
            for k in range(EMBED_DIM // LANES):
                sl = pl.ds(k * LANES, LANES)
                a = rows_in[r, sl]
                p = rows_pos[r, sl]
                n = rows_neg[r, sl]
                accp = accp + a * p
                accn = accn + a * n
            pacc_v[pl.ds(i * LANES, LANES)] = accp
            nacc_v[pl.ds(i * LANES, LANES)] = accn
        totp = jnp.zeros((LANES,), jnp.float32)
        totn = jnp.zeros((LANES,), jnp.float32)
        for d in range(LANES):
            idx = iota16 * LANES + d
            totp = totp + plsc.load_gather(pacc_v, [idx])
            totn = totn + plsc.load_gather(nacc_v, [idx])
        score_pos[pl.ds(c * LANES, LANES)] = totp
        score_neg[pl.ds(c * LANES, LANES)] = totn
        return carry

    lax.fori_loop(0, ROWS_PER_WORKER // LANES, chunk_body, 0)

    base = wid * ROWS_PER_WORKER
    pltpu.sync_copy(score_pos, pos_out.at[pl.ds(base, ROWS_PER_WORKER)])
    pltpu.sync_copy(score_neg, neg_out.at[pl.ds(base, ROWS_PER_WORKER)])


@functools.partial(jax.jit, static_argnames=())
def _skipgram_scores(in_embed, out_embed, idx_in, idx_pos, idx_neg):
    batch = idx_in.shape[0] * idx_in.shape[1]
    mesh = plsc.VectorSubcoreMesh(
        core_axis_name="c", subcore_axis_name="s",
        num_cores=NUM_CORES, num_subcores=NUM_SUBCORES)
    run = pl.kernel(
        _sc_body,
        out_type=(
            jax.ShapeDtypeStruct((batch,), jnp.float32),
            jax.ShapeDtypeStruct((batch,), jnp.float32),
        ),
        mesh=mesh,
        scratch_types=[
            pltpu.VMEM((N_CHUNKS, CHUNK), jnp.int32),
            pltpu.VMEM((N_CHUNKS, CHUNK), jnp.int32),
            pltpu.VMEM((N_CHUNKS, CHUNK), jnp.int32),
            pltpu.VMEM((ROWS_PER_WORKER, EMBED_DIM), jnp.float32),
            pltpu.VMEM((ROWS_PER_WORKER, EMBED_DIM), jnp.float32),
            pltpu.VMEM((ROWS_PER_WORKER, EMBED_DIM), jnp.float32),
            pltpu.VMEM((LANES * LANES,), jnp.float32),
            pltpu.VMEM((LANES * LANES,), jnp.float32),
            pltpu.VMEM((ROWS_PER_WORKER,), jnp.float32),
            pltpu.VMEM((ROWS_PER_WORKER,), jnp.float32),
            pltpu.SemaphoreType.DMA,
        ],
        compiler_params=pltpu.CompilerParams(
            needs_layout_passes=False, use_tc_tiling_on_sc=False),
    )
    return run(in_embed, out_embed, idx_in, idx_pos, idx_neg)


def kernel(input_labels, pos_labels, neg_labels, in_embed, out_embed):
    batch = input_labels.shape[0]
    idx_in = input_labels.astype(jnp.int32).reshape(batch // CHUNK, CHUNK)
    idx_pos = pos_labels.astype(jnp.int32).reshape(batch // CHUNK, CHUNK)
    idx_neg = neg_labels.astype(jnp.int32).reshape(batch // CHUNK, CHUNK)
    pos_score, neg_score = _skipgram_scores(
        in_embed, out_embed, idx_in, idx_pos, idx_neg)
    return pos_score, neg_score.reshape(batch, 1)
